# Initial kernel scaffold; baseline (speedup 1.0000x reference)
import jax
import jax.numpy as jnp
from jax.experimental import pallas as pl


def kernel(act, location, duration, tst, tet, travel, edge_index, batch, node_emb0, node_emb1, edge_emb0, W, att_src, att_dst, W_edge, att_edge, gat_bias, fc_W, fc_b):
    raise NotImplementedError("write your pallas kernel here")


# speedup vs baseline: 6.0713x; 6.0713x over previous
import functools

import jax
import jax.numpy as jnp
from jax import lax
from jax.experimental import pallas as pl
from jax.experimental.pallas import tpu as pltpu
from jax.experimental.pallas import tpu_sc as plsc

N_NODES = 100000
N_EDGES = 1600000
HIDDEN = 32
N_GRAPHS = 256
TARGET = 16

NODE_BLK = 2000
EDGE_ROWS = 3125
EDGE_COLS = 512
EDGE_BLK = 128
D_BLK = 6400


def _node_body(act_ref, loc_ref, e0_ref, e1_ref, w_ref, asv_ref, adv_ref,
               h_ref, asrc_ref, adst_ref):
    act = act_ref[...]
    loc = loc_ref[...]
    i100 = lax.broadcasted_iota(jnp.int32, (NODE_BLK, 100), 1)
    i1000 = lax.broadcasted_iota(jnp.int32, (NODE_BLK, 1000), 1)
    oh0 = (act == i100).astype(jnp.float32)
    oh1 = (loc == i1000).astype(jnp.float32)
    x = jnp.dot(oh0, e0_ref[...], preferred_element_type=jnp.float32)
    x = x + jnp.dot(oh1, e1_ref[...], preferred_element_type=jnp.float32)
    x = jnp.maximum(x, 0.0)
    h = jnp.dot(x, w_ref[...], preferred_element_type=jnp.float32)
    h_ref[...] = h
    asrc_ref[...] = jnp.dot(h, asv_ref[...], preferred_element_type=jnp.float32)
    adst_ref[...] = jnp.dot(h, adv_ref[...], preferred_element_type=jnp.float32)


def _node_stage(act, loc, e0, e1, W, att_src, att_dst):
    grid = N_NODES // NODE_BLK
    return pl.pallas_call(
        _node_body,
        grid=(grid,),
        in_specs=[
            pl.BlockSpec((NODE_BLK, 1), lambda i: (i, 0)),
            pl.BlockSpec((NODE_BLK, 1), lambda i: (i, 0)),
            pl.BlockSpec((100, HIDDEN), lambda i: (0, 0)),
            pl.BlockSpec((1000, HIDDEN), lambda i: (0, 0)),
            pl.BlockSpec((HIDDEN, HIDDEN), lambda i: (0, 0)),
            pl.BlockSpec((HIDDEN, 1), lambda i: (0, 0)),
            pl.BlockSpec((HIDDEN, 1), lambda i: (0, 0)),
        ],
        out_specs=[
            pl.BlockSpec((NODE_BLK, HIDDEN), lambda i: (i, 0)),
            pl.BlockSpec((NODE_BLK, 1), lambda i: (i, 0)),
            pl.BlockSpec((NODE_BLK, 1), lambda i: (i, 0)),
        ],
        out_shape=[
            jax.ShapeDtypeStruct((N_NODES, HIDDEN), jnp.float32),
            jax.ShapeDtypeStruct((N_NODES, 1), jnp.float32),
            jax.ShapeDtypeStruct((N_NODES, 1), jnp.float32),
        ],
    )(act, loc, e0, e1, W, att_src, att_dst)


def _edge_body2(travel_ref, dur_ref, tst_ref, tet_ref, w10_ref, cvec_ref, out_ref):
    t = travel_ref[...]
    acc = jnp.zeros(t.shape, jnp.float32)
    for k in range(10):
        acc = acc + jnp.where(t == k, w10_ref[k, 0], 0.0)
    out_ref[...] = (acc + dur_ref[...] * cvec_ref[0, 0]
                    + tst_ref[...] * cvec_ref[1, 0]
                    + tet_ref[...] * cvec_ref[2, 0])


def _edge_stage(travel2, dur2, tst2, tet2, w10, cvec):
    grid = EDGE_COLS // EDGE_BLK
    return pl.pallas_call(
        _edge_body2,
        grid=(grid,),
        in_specs=[
            pl.BlockSpec((EDGE_ROWS, EDGE_BLK), lambda i: (0, i)),
            pl.BlockSpec((EDGE_ROWS, EDGE_BLK), lambda i: (0, i)),
            pl.BlockSpec((EDGE_ROWS, EDGE_BLK), lambda i: (0, i)),
            pl.BlockSpec((EDGE_ROWS, EDGE_BLK), lambda i: (0, i)),
            pl.BlockSpec((10, 1), lambda i: (0, 0)),
            pl.BlockSpec((3, 1), lambda i: (0, 0)),
        ],
        out_specs=pl.BlockSpec((EDGE_ROWS, EDGE_BLK), lambda i: (0, i)),
        out_shape=jax.ShapeDtypeStruct((EDGE_ROWS, EDGE_COLS), jnp.float32),
    )(travel2, dur2, tst2, tet2, w10, cvec)


def _proj_body(we_ref, aev_ref, e0_ref, w10_ref, cvec_ref):
    v = jnp.dot(we_ref[...], aev_ref[...], preferred_element_type=jnp.float32)
    w10_ref[...] = jnp.dot(e0_ref[...], v[:HIDDEN],
                           preferred_element_type=jnp.float32)
    cvec_ref[...] = v[HIDDEN:HIDDEN + 3]


def _proj_stage(W_edge, att_edge, edge_emb0):
    return pl.pallas_call(
        _proj_body,
        out_shape=[
            jax.ShapeDtypeStruct((10, 1), jnp.float32),
            jax.ShapeDtypeStruct((3, 1), jnp.float32),
        ],
    )(W_edge, att_edge, edge_emb0)


CH = 128
N_CHUNKS = N_EDGES // CH


def _make_gather():
    info = plsc.get_sparse_core_info()
    NC, NS = info.num_cores, info.num_subcores
    NW = NC * NS
    per_w = (N_CHUNKS + NW - 1) // NW
    mesh = plsc.VectorSubcoreMesh(core_axis_name="c", subcore_axis_name="s")

    @functools.partial(
        pl.kernel, mesh=mesh,
        compiler_params=pltpu.CompilerParams(use_tc_tiling_on_sc=False),
        out_type=[
            jax.ShapeDtypeStruct((N_EDGES, 48), jnp.float32),
            jax.ShapeDtypeStruct((N_EDGES, 16), jnp.float32),
        ],
        scratch_types=[
            pltpu.VMEM((CH,), jnp.int32),
            pltpu.VMEM((CH,), jnp.int32),
            pltpu.VMEM((CH, 48), jnp.float32),
            pltpu.VMEM((CH, 16), jnp.float32),
            pltpu.SemaphoreType.DMA,
            pltpu.SemaphoreType.DMA,
        ],
    )
    def gather_k(t48_hbm, td_hbm, src_hbm, dst_hbm, o1_hbm, o2_hbm,
                 sidx, didx, r1, r2, sem1, sem2):
        wid = lax.axis_index("s") * NC + lax.axis_index("c")

        def body(j, carry):
            chunk = wid + j * NW

            @pl.when(chunk < N_CHUNKS)
            def _():
                off = chunk * CH
                pltpu.sync_copy(src_hbm.at[pl.ds(off, CH)], sidx)
                pltpu.sync_copy(dst_hbm.at[pl.ds(off, CH)], didx)
                c1 = pltpu.async_copy(t48_hbm.at[sidx], r1, sem1)
                c2 = pltpu.async_copy(td_hbm.at[didx], r2, sem2)
                c1.wait()
                c2.wait()
                pltpu.sync_copy(r1, o1_hbm.at[pl.ds(off, CH)])
                pltpu.sync_copy(r2, o2_hbm.at[pl.ds(off, CH)])

            return carry

        lax.fori_loop(0, per_w, body, 0)

    return gather_k


def _msg_body(g1_ref, g2_ref, ae_ref, c_ref, msg_ref, ex_ref):
    g1 = g1_ref[...]
    a = g1[:, HIDDEN:HIDDEN + 1] + g2_ref[...][:, 0:1] + ae_ref[...]
    a = jnp.where(a >= 0.0, a, 0.2 * a)
    ex = jnp.exp(a - c_ref[0, 0])
    ex_ref[...] = ex
    msg_ref[...] = g1[:, :HIDDEN] * ex


def _msg_stage(g1, g2, ae, cval):
    grid = N_EDGES // D_BLK
    return pl.pallas_call(
        _msg_body,
        grid=(grid,),
        in_specs=[
            pl.BlockSpec((D_BLK, 48), lambda i: (i, 0)),
            pl.BlockSpec((D_BLK, 16), lambda i: (i, 0)),
            pl.BlockSpec((D_BLK, 1), lambda i: (i, 0)),
            pl.BlockSpec((1, 1), lambda i: (0, 0)),
        ],
        out_specs=[
            pl.BlockSpec((D_BLK, HIDDEN), lambda i: (i, 0)),
            pl.BlockSpec((D_BLK, 1), lambda i: (i, 0)),
        ],
        out_shape=[
            jax.ShapeDtypeStruct((N_EDGES, HIDDEN), jnp.float32),
            jax.ShapeDtypeStruct((N_EDGES, 1), jnp.float32),
        ],
    )(g1, g2, ae, cval)


def _fin_body(num_ref, den_ref, batch_ref, bias_ref, fcw_ref, fcb_ref,
              out_ref, gsum, gcnt):
    i = pl.program_id(0)
    n = pl.num_programs(0)

    @pl.when(i == 0)
    def _():
        gsum[...] = jnp.zeros_like(gsum)
        gcnt[...] = jnp.zeros_like(gcnt)

    node = jnp.maximum(num_ref[...] / (den_ref[...] + 1e-16) + bias_ref[...], 0.0)
    b = batch_ref[...]
    ig = lax.broadcasted_iota(jnp.int32, (NODE_BLK, N_GRAPHS), 1)
    oh = (b == ig).astype(jnp.float32)
    gsum[...] += lax.dot_general(oh, node, (((0,), (0,)), ((), ())),
                                 preferred_element_type=jnp.float32)
    gcnt[...] += jnp.sum(oh, axis=0, keepdims=True)

    @pl.when(i == n - 1)
    def _():
        cnt = jnp.maximum(gcnt[...], 1.0)
        gmean = gsum[...] / cnt.reshape(N_GRAPHS, 1)
        logits = jnp.dot(gmean, fcw_ref[...],
                         preferred_element_type=jnp.float32) + fcb_ref[...]
        m = jnp.max(logits, axis=1, keepdims=True)
        z = logits - m
        out_ref[...] = z - jnp.log(jnp.sum(jnp.exp(z), axis=1, keepdims=True))


def _fin_stage(num, den, batch, bias, fc_W, fc_b):
    grid = N_NODES // NODE_BLK
    return pl.pallas_call(
        _fin_body,
        grid=(grid,),
        in_specs=[
            pl.BlockSpec((NODE_BLK, HIDDEN), lambda i: (i, 0)),
            pl.BlockSpec((NODE_BLK, 1), lambda i: (i, 0)),
            pl.BlockSpec((NODE_BLK, 1), lambda i: (i, 0)),
            pl.BlockSpec((1, HIDDEN), lambda i: (0, 0)),
            pl.BlockSpec((HIDDEN, TARGET), lambda i: (0, 0)),
            pl.BlockSpec((1, TARGET), lambda i: (0, 0)),
        ],
        out_specs=pl.BlockSpec((N_GRAPHS, TARGET), lambda i: (0, 0)),
        out_shape=jax.ShapeDtypeStruct((N_GRAPHS, TARGET), jnp.float32),
        scratch_shapes=[
            pltpu.VMEM((N_GRAPHS, HIDDEN), jnp.float32),
            pltpu.VMEM((1, N_GRAPHS), jnp.float32),
        ],
    )(num, den, batch, bias, fc_W, fc_b)


def kernel(act, location, duration, tst, tet, travel, edge_index, batch,
           node_emb0, node_emb1, edge_emb0, W, att_src, att_dst,
           W_edge, att_edge, gat_bias, fc_W, fc_b):
    act = act.astype(jnp.int32).reshape(N_NODES, 1)
    location = location.astype(jnp.int32).reshape(N_NODES, 1)
    src = edge_index[0].astype(jnp.int32)
    dst = edge_index[1].astype(jnp.int32)

    h, asrc, adst = _node_stage(act, location, node_emb0, node_emb1, W,
                                att_src.reshape(HIDDEN, 1),
                                att_dst.reshape(HIDDEN, 1))

    w10, cvec = _proj_stage(W_edge, att_edge.reshape(HIDDEN, 1), edge_emb0)
    ae2 = _edge_stage(travel.astype(jnp.int32).reshape(EDGE_ROWS, EDGE_COLS),
                      duration.reshape(EDGE_ROWS, EDGE_COLS),
                      tst.reshape(EDGE_ROWS, EDGE_COLS),
                      tet.reshape(EDGE_ROWS, EDGE_COLS), w10, cvec)
    ae = ae2.reshape(N_EDGES, 1)

    cval = jnp.maximum(jnp.max(asrc) + jnp.max(adst) + jnp.max(ae), 0.0)
    cval = cval.reshape(1, 1)

    t48 = jnp.concatenate(
        [h, asrc, jnp.zeros((N_NODES, 15), jnp.float32)], axis=1)
    td16 = jnp.concatenate(
        [adst, jnp.zeros((N_NODES, 15), jnp.float32)], axis=1)

    g1, g2 = _make_gather()(t48, td16, src, dst)

    msg, ex = _msg_stage(g1, g2, ae, cval)

    num = jax.ops.segment_sum(msg, dst, num_segments=N_NODES)
    den = jax.ops.segment_sum(ex[:, 0], dst, num_segments=N_NODES)

    return _fin_stage(num, den.reshape(N_NODES, 1),
                      batch.astype(jnp.int32).reshape(N_NODES, 1),
                      gat_bias.reshape(1, HIDDEN), fc_W,
                      fc_b.reshape(1, TARGET))
